# baseline (speedup 1.0000x reference)
import functools

import jax
import jax.numpy as jnp
from jax import lax
from jax.experimental import pallas as pl
from jax.experimental.pallas import tpu as pltpu
from jax.experimental.pallas import tpu_sc as plsc

NC = 2
NS = 16
NW = NC * NS
CHUNK = 128
NBUF = 3


@functools.lru_cache(maxsize=None)
def _make_sc_gather(B, V, D):
    b_per_w = B // NW
    n_chunks = b_per_w // CHUNK
    nbuf = min(NBUF, n_chunks)
    mesh = plsc.VectorSubcoreMesh(core_axis_name="c", subcore_axis_name="s")

    @functools.partial(
        pl.kernel,
        mesh=mesh,
        out_type=jax.ShapeDtypeStruct((B, D), jnp.float32),
        scratch_types=(
            [pltpu.VMEM((b_per_w,), jnp.int32)]
            + [pltpu.VMEM((CHUNK, D), jnp.float32) for _ in range(nbuf)]
            + [pltpu.SemaphoreType.DMA for _ in range(2 * nbuf)]
        ),
    )
    def gather_kernel(idx_hbm, table_hbm, out_hbm, idx_v, *refs):
        rows = list(refs[:nbuf])
        gsem = list(refs[nbuf:2 * nbuf])
        wsem = list(refs[2 * nbuf:3 * nbuf])

        wid = lax.axis_index("s") * NC + lax.axis_index("c")
        base = wid * b_per_w

        pltpu.sync_copy(idx_hbm.at[pl.ds(base, b_per_w)], idx_v)

        def fire_gather(c):
            return pltpu.async_copy(
                table_hbm.at[idx_v.at[pl.ds(c * CHUNK, CHUNK)]],
                rows[c % nbuf], gsem[c % nbuf])

        gcp = [None] * nbuf
        wcp = [None] * n_chunks
        w_done = [False] * n_chunks
        for c in range(nbuf):
            gcp[c % nbuf] = fire_gather(c)
        for c in range(n_chunks):
            gcp[c % nbuf].wait()
            wcp[c] = pltpu.async_copy(
                rows[c % nbuf],
                out_hbm.at[pl.ds(base + c * CHUNK, CHUNK)],
                wsem[c % nbuf])
            if c + nbuf < n_chunks:
                wcp[c].wait()
                w_done[c] = True
                gcp[c % nbuf] = fire_gather(c + nbuf)
        for c in range(n_chunks):
            if not w_done[c]:
                wcp[c].wait()

    return gather_kernel


@functools.lru_cache(maxsize=None)
def _make_tc_proj(B, D, E, blk):
    def body(x_ref, w_ref, b_ref, o_ref):
        o_ref[...] = lax.dot_general(
            x_ref[...], w_ref[...],
            (((1,), (1,)), ((), ())),
            preferred_element_type=jnp.float32,
        ) + b_ref[...]

    return pl.pallas_call(
        body,
        grid=(B // blk,),
        in_specs=[
            pl.BlockSpec((blk, D), lambda i: (i, 0)),
            pl.BlockSpec((E, D), lambda i: (0, 0)),
            pl.BlockSpec((1, E), lambda i: (0, 0)),
        ],
        out_specs=pl.BlockSpec((blk, E), lambda i: (i, 0)),
        out_shape=jax.ShapeDtypeStruct((B, E), jnp.float32),
    )


def kernel(t, pos_enc, W, b):
    B = t.shape[0]
    V, D = pos_enc.shape
    E = W.shape[0]

    embed = _make_sc_gather(B, V, D)(t, pos_enc)
    proj = _make_tc_proj(B, D, E, 1024)
    return proj(embed, W, b.reshape(1, E))


# speedup vs baseline: 1.1636x; 1.1636x over previous
import functools

import jax
import jax.numpy as jnp
from jax import lax
from jax.experimental import pallas as pl
from jax.experimental.pallas import tpu as pltpu
from jax.experimental.pallas import tpu_sc as plsc

NC = 2
NS = 16
NW = NC * NS
CHUNK = 128
NBUF = 3


@functools.lru_cache(maxsize=None)
def _make_sc_gather(B, V, D):
    b_per_w = B // NW
    n_chunks = b_per_w // CHUNK
    nbuf = min(NBUF, n_chunks)
    mesh = plsc.VectorSubcoreMesh(core_axis_name="c", subcore_axis_name="s")

    @functools.partial(
        pl.kernel,
        mesh=mesh,
        out_type=jax.ShapeDtypeStruct((B, D), jnp.float32),
        scratch_types=(
            [pltpu.VMEM((b_per_w,), jnp.int32)]
            + [pltpu.VMEM((CHUNK, D), jnp.float32) for _ in range(nbuf)]
            + [pltpu.SemaphoreType.DMA for _ in range(2 * nbuf)]
        ),
    )
    def gather_kernel(idx_hbm, table_hbm, out_hbm, idx_v, *refs):
        rows = list(refs[:nbuf])
        gsem = list(refs[nbuf:2 * nbuf])
        wsem = list(refs[2 * nbuf:3 * nbuf])

        wid = lax.axis_index("s") * NC + lax.axis_index("c")
        base = wid * b_per_w

        pltpu.sync_copy(idx_hbm.at[pl.ds(base, b_per_w)], idx_v)

        def fire_gather(c):
            return pltpu.async_copy(
                table_hbm.at[idx_v.at[pl.ds(c * CHUNK, CHUNK)]],
                rows[c % nbuf], gsem[c % nbuf])

        gcp = [None] * nbuf
        wcp = [None] * n_chunks
        w_done = [False] * n_chunks
        for c in range(nbuf):
            gcp[c % nbuf] = fire_gather(c)
        for c in range(n_chunks):
            gcp[c % nbuf].wait()
            wcp[c] = pltpu.async_copy(
                rows[c % nbuf],
                out_hbm.at[pl.ds(base + c * CHUNK, CHUNK)],
                wsem[c % nbuf])
            if c + nbuf < n_chunks:
                wcp[c].wait()
                w_done[c] = True
                gcp[c % nbuf] = fire_gather(c + nbuf)
        for c in range(n_chunks):
            if not w_done[c]:
                wcp[c].wait()

    return gather_kernel


@functools.lru_cache(maxsize=None)
def _make_tc_proj(B, D, E, blk):
    def body(x_ref, w_ref, b_ref, o_ref):
        o_ref[...] = lax.dot_general(
            x_ref[...], w_ref[...],
            (((1,), (1,)), ((), ())),
            preferred_element_type=jnp.float32,
        ) + b_ref[...]

    return pl.pallas_call(
        body,
        grid=(B // blk,),
        in_specs=[
            pl.BlockSpec((blk, D), lambda i: (i, 0)),
            pl.BlockSpec((E, D), lambda i: (0, 0)),
            pl.BlockSpec((1, E), lambda i: (0, 0)),
        ],
        out_specs=pl.BlockSpec((blk, E), lambda i: (i, 0)),
        out_shape=jax.ShapeDtypeStruct((B, E), jnp.float32),
    )


def kernel(t, pos_enc, W, b):
    B = t.shape[0]
    V, D = pos_enc.shape
    E = W.shape[0]

    embed = _make_sc_gather(B, V, D)(t, pos_enc)
    proj = _make_tc_proj(B, D, E, 4096)
    return proj(embed, W, b.reshape(1, E))


# speedup vs baseline: 1.1891x; 1.0220x over previous
import functools

import jax
import jax.numpy as jnp
from jax import lax
from jax.experimental import pallas as pl
from jax.experimental.pallas import tpu as pltpu
from jax.experimental.pallas import tpu_sc as plsc

NC = 2
NS = 16
NW = NC * NS
CHUNK = 128
NBUF = 3


@functools.lru_cache(maxsize=None)
def _make_sc_gather(B, V, D):
    b_per_w = B // NW
    n_chunks = b_per_w // CHUNK
    nbuf = min(NBUF, n_chunks)
    mesh = plsc.VectorSubcoreMesh(core_axis_name="c", subcore_axis_name="s")

    @functools.partial(
        pl.kernel,
        mesh=mesh,
        out_type=jax.ShapeDtypeStruct((B, D), jnp.float32),
        scratch_types=(
            [pltpu.VMEM((b_per_w,), jnp.int32)]
            + [pltpu.VMEM((CHUNK, D), jnp.float32) for _ in range(nbuf)]
            + [pltpu.SemaphoreType.DMA for _ in range(2 * nbuf)]
        ),
    )
    def gather_kernel(idx_hbm, table_hbm, out_hbm, idx_v, *refs):
        rows = list(refs[:nbuf])
        gsem = list(refs[nbuf:2 * nbuf])
        wsem = list(refs[2 * nbuf:3 * nbuf])

        wid = lax.axis_index("s") * NC + lax.axis_index("c")
        base = wid * b_per_w

        pltpu.sync_copy(idx_hbm.at[pl.ds(base, b_per_w)], idx_v)

        def fire_gather(c):
            return pltpu.async_copy(
                table_hbm.at[idx_v.at[pl.ds(c * CHUNK, CHUNK)]],
                rows[c % nbuf], gsem[c % nbuf])

        gcp = [None] * nbuf
        wcp = [None] * n_chunks
        w_done = [False] * n_chunks
        for c in range(nbuf):
            gcp[c % nbuf] = fire_gather(c)
        for c in range(n_chunks):
            gcp[c % nbuf].wait()
            wcp[c] = pltpu.async_copy(
                rows[c % nbuf],
                out_hbm.at[pl.ds(base + c * CHUNK, CHUNK)],
                wsem[c % nbuf])
            if c + nbuf < n_chunks:
                wcp[c].wait()
                w_done[c] = True
                gcp[c % nbuf] = fire_gather(c + nbuf)
        for c in range(n_chunks):
            if not w_done[c]:
                wcp[c].wait()

    return gather_kernel


@functools.lru_cache(maxsize=None)
def _make_tc_proj(B, D, E, blk):
    def body(x_ref, w_ref, b_ref, o_ref):
        o_ref[...] = lax.dot_general(
            x_ref[...], w_ref[...],
            (((1,), (1,)), ((), ())),
            preferred_element_type=jnp.float32,
        ) + b_ref[...]

    return pl.pallas_call(
        body,
        grid=(B // blk,),
        in_specs=[
            pl.BlockSpec((blk, D), lambda i: (i, 0)),
            pl.BlockSpec((E, D), lambda i: (0, 0)),
            pl.BlockSpec((1, E), lambda i: (0, 0)),
        ],
        out_specs=pl.BlockSpec((blk, E), lambda i: (i, 0)),
        out_shape=jax.ShapeDtypeStruct((B, E), jnp.float32),
    )


def kernel(t, pos_enc, W, b):
    B = t.shape[0]
    V, D = pos_enc.shape
    E = W.shape[0]

    embed = _make_sc_gather(B, V, D)(t, pos_enc)
    proj = _make_tc_proj(B, D, E, 8192)
    return proj(embed, W, b.reshape(1, E))


# speedup vs baseline: 1.2131x; 1.0202x over previous
import functools

import jax
import jax.numpy as jnp
from jax import lax
from jax.experimental import pallas as pl
from jax.experimental.pallas import tpu as pltpu
from jax.experimental.pallas import tpu_sc as plsc

NC = 2
NS = 16
NW = NC * NS
CHUNK = 64
NBUF = 6


@functools.lru_cache(maxsize=None)
def _make_sc_gather(B, V, D):
    b_per_w = B // NW
    n_chunks = b_per_w // CHUNK
    nbuf = min(NBUF, n_chunks)
    mesh = plsc.VectorSubcoreMesh(core_axis_name="c", subcore_axis_name="s")

    @functools.partial(
        pl.kernel,
        mesh=mesh,
        out_type=jax.ShapeDtypeStruct((B, D), jnp.float32),
        scratch_types=(
            [pltpu.VMEM((b_per_w,), jnp.int32)]
            + [pltpu.VMEM((CHUNK, D), jnp.float32) for _ in range(nbuf)]
            + [pltpu.SemaphoreType.DMA for _ in range(2 * nbuf)]
        ),
    )
    def gather_kernel(idx_hbm, table_hbm, out_hbm, idx_v, *refs):
        rows = list(refs[:nbuf])
        gsem = list(refs[nbuf:2 * nbuf])
        wsem = list(refs[2 * nbuf:3 * nbuf])

        wid = lax.axis_index("s") * NC + lax.axis_index("c")
        base = wid * b_per_w

        pltpu.sync_copy(idx_hbm.at[pl.ds(base, b_per_w)], idx_v)

        def fire_gather(c):
            return pltpu.async_copy(
                table_hbm.at[idx_v.at[pl.ds(c * CHUNK, CHUNK)]],
                rows[c % nbuf], gsem[c % nbuf])

        gcp = [None] * nbuf
        wcp = [None] * n_chunks
        w_done = [False] * n_chunks
        for c in range(nbuf):
            gcp[c % nbuf] = fire_gather(c)
        for c in range(n_chunks):
            gcp[c % nbuf].wait()
            wcp[c] = pltpu.async_copy(
                rows[c % nbuf],
                out_hbm.at[pl.ds(base + c * CHUNK, CHUNK)],
                wsem[c % nbuf])
            if c + nbuf < n_chunks:
                wcp[c].wait()
                w_done[c] = True
                gcp[c % nbuf] = fire_gather(c + nbuf)
        for c in range(n_chunks):
            if not w_done[c]:
                wcp[c].wait()

    return gather_kernel


@functools.lru_cache(maxsize=None)
def _make_tc_proj(B, D, E, blk):
    def body(x_ref, w_ref, b_ref, o_ref):
        o_ref[...] = lax.dot_general(
            x_ref[...], w_ref[...],
            (((1,), (1,)), ((), ())),
            preferred_element_type=jnp.float32,
        ) + b_ref[...]

    return pl.pallas_call(
        body,
        grid=(B // blk,),
        in_specs=[
            pl.BlockSpec((blk, D), lambda i: (i, 0)),
            pl.BlockSpec((E, D), lambda i: (0, 0)),
            pl.BlockSpec((1, E), lambda i: (0, 0)),
        ],
        out_specs=pl.BlockSpec((blk, E), lambda i: (i, 0)),
        out_shape=jax.ShapeDtypeStruct((B, E), jnp.float32),
    )


def kernel(t, pos_enc, W, b):
    B = t.shape[0]
    V, D = pos_enc.shape
    E = W.shape[0]

    embed = _make_sc_gather(B, V, D)(t, pos_enc)
    proj = _make_tc_proj(B, D, E, 8192)
    return proj(embed, W, b.reshape(1, E))
